# baseline (speedup 1.0000x reference)
import jax
import jax.numpy as jnp
from jax.experimental import pallas as pl
from jax.experimental.pallas import tpu as pltpu

E = 16
TILE = 2048
D = 2048


def _router_body(x_ref, w_ref, weights_ref, topw_ref, tope_ref):
    x = x_ref[...]
    w = w_ref[...]
    lt = jax.lax.dot_general(w, x, (((0,), (1,)), ((), ())),
                             preferred_element_type=jnp.float32)
    m = jnp.max(lt, axis=0, keepdims=True)
    ex = jnp.exp(lt - m)
    s = jnp.sum(ex, axis=0, keepdims=True)
    weights_ref[...] = (ex / s).T

    row = jax.lax.broadcasted_iota(jnp.int32, lt.shape, 0)
    i1 = jnp.min(jnp.where(lt == m, row, E), axis=0, keepdims=True)
    masked = jnp.where(row == i1, -jnp.inf, lt)
    l2 = jnp.max(masked, axis=0, keepdims=True)
    i2 = jnp.min(jnp.where(masked == l2, row, E), axis=0, keepdims=True)
    e2 = jnp.exp(l2 - m)
    r = 1.0 / (1.0 + e2)
    topw_ref[...] = jnp.concatenate([r, e2 * r], axis=0).T
    tope_ref[...] = jnp.concatenate([i1, i2], axis=0).T


def kernel(x, W):
    B, S, _ = x.shape
    N = B * S
    x2 = x.reshape(N, D)
    grid = (N // TILE,)
    weights, topw, tope = pl.pallas_call(
        _router_body,
        grid=grid,
        in_specs=[
            pl.BlockSpec((TILE, D), lambda i: (i, 0)),
            pl.BlockSpec((D, E), lambda i: (0, 0)),
        ],
        out_specs=[
            pl.BlockSpec((TILE, E), lambda i: (i, 0)),
            pl.BlockSpec((TILE, 2), lambda i: (i, 0)),
            pl.BlockSpec((TILE, 2), lambda i: (i, 0)),
        ],
        out_shape=[
            jax.ShapeDtypeStruct((N, E), jnp.float32),
            jax.ShapeDtypeStruct((N, 2), jnp.float32),
            jax.ShapeDtypeStruct((N, 2), jnp.int32),
        ],
    )(x2, W)
    return (
        weights.reshape(B, S, E),
        topw.reshape(B, S, 2),
        tope.reshape(B, S, 2),
    )


# speedup vs baseline: 1.0122x; 1.0122x over previous
import jax
import jax.numpy as jnp
from jax.experimental import pallas as pl
from jax.experimental.pallas import tpu as pltpu

E = 16
TILE = 512
D = 2048
NBUF = 6


def _copy_in(x_hbm, xbuf, insem, g, slot):
    return pltpu.make_async_copy(
        x_hbm.at[pl.ds(g * TILE, TILE), :], xbuf.at[slot], insem.at[slot])


def _router_body(x_hbm, w_ref, weights_ref, topw_ref, tope_ref, xbuf, insem):
    g = pl.program_id(0)
    nch = pl.num_programs(0)

    @pl.when(g == 0)
    def _prime():
        for s in range(NBUF - 1):
            _copy_in(x_hbm, xbuf, insem, s, s).start()

    nxt = g + NBUF - 1

    @pl.when(nxt < nch)
    def _prefetch():
        _copy_in(x_hbm, xbuf, insem, nxt, jax.lax.rem(nxt, NBUF)).start()

    slot = jax.lax.rem(g, NBUF)
    _copy_in(x_hbm, xbuf, insem, g, slot).wait()

    x = xbuf[slot]
    w = w_ref[...]
    lt = jax.lax.dot_general(w, x, (((0,), (1,)), ((), ())),
                             preferred_element_type=jnp.float32)
    m = jnp.max(lt, axis=0, keepdims=True)
    ex = jnp.exp(lt - m)
    s = jnp.sum(ex, axis=0, keepdims=True)
    weights_ref[...] = (ex / s).T

    row = jax.lax.broadcasted_iota(jnp.int32, lt.shape, 0)
    i1 = jnp.min(jnp.where(lt == m, row, E), axis=0, keepdims=True)
    masked = jnp.where(row == i1, -jnp.inf, lt)
    l2 = jnp.max(masked, axis=0, keepdims=True)
    i2 = jnp.min(jnp.where(masked == l2, row, E), axis=0, keepdims=True)
    e2 = jnp.exp(l2 - m)
    r = 1.0 / (1.0 + e2)
    topw_ref[...] = jnp.concatenate([r, e2 * r], axis=0).T
    tope_ref[...] = jnp.concatenate([i1, i2], axis=0).T


def kernel(x, W):
    B, S, _ = x.shape
    N = B * S
    x2 = x.reshape(N, D)
    grid = (N // TILE,)
    weights, topw, tope = pl.pallas_call(
        _router_body,
        grid=grid,
        in_specs=[
            pl.BlockSpec(memory_space=pl.ANY),
            pl.BlockSpec((D, E), lambda i: (0, 0)),
        ],
        out_specs=[
            pl.BlockSpec((TILE, E), lambda i: (i, 0)),
            pl.BlockSpec((TILE, 2), lambda i: (i, 0)),
            pl.BlockSpec((TILE, 2), lambda i: (i, 0)),
        ],
        out_shape=[
            jax.ShapeDtypeStruct((N, E), jnp.float32),
            jax.ShapeDtypeStruct((N, 2), jnp.float32),
            jax.ShapeDtypeStruct((N, 2), jnp.int32),
        ],
        scratch_shapes=[
            pltpu.VMEM((NBUF, TILE, D), jnp.float32),
            pltpu.SemaphoreType.DMA((NBUF,)),
        ],
    )(x2, W)
    return (
        weights.reshape(B, S, E),
        topw.reshape(B, S, 2),
        tope.reshape(B, S, 2),
    )


# speedup vs baseline: 1.0219x; 1.0096x over previous
import jax
import jax.numpy as jnp
from jax.experimental import pallas as pl
from jax.experimental.pallas import tpu as pltpu

E = 16
TILE = 512
D = 2048
NSTREAM = 2


def _route_one(x, w, weights_ref, topw_ref, tope_ref):
    lt = jax.lax.dot_general(w, x, (((0,), (1,)), ((), ())),
                             preferred_element_type=jnp.float32)
    m = jnp.max(lt, axis=0, keepdims=True)
    ex = jnp.exp(lt - m)
    s = jnp.sum(ex, axis=0, keepdims=True)
    weights_ref[...] = (ex / s).T

    row = jax.lax.broadcasted_iota(jnp.int32, lt.shape, 0)
    i1 = jnp.min(jnp.where(lt == m, row, E), axis=0, keepdims=True)
    masked = jnp.where(row == i1, -jnp.inf, lt)
    l2 = jnp.max(masked, axis=0, keepdims=True)
    i2 = jnp.min(jnp.where(masked == l2, row, E), axis=0, keepdims=True)
    e2 = jnp.exp(l2 - m)
    r = 1.0 / (1.0 + e2)
    topw_ref[...] = jnp.concatenate([r, e2 * r], axis=0).T
    tope_ref[...] = jnp.concatenate([i1, i2], axis=0).T


def _router_body(*refs):
    x_refs = refs[:NSTREAM]
    w = refs[NSTREAM][...]
    out = refs[NSTREAM + 1:]
    for s in range(NSTREAM):
        _route_one(x_refs[s][...], w, out[3 * s], out[3 * s + 1], out[3 * s + 2])


def kernel(x, W):
    B, S, _ = x.shape
    N = B * S
    M = N // NSTREAM
    nch = M // TILE
    x2 = x.reshape(N, D)

    in_specs = [
        pl.BlockSpec((TILE, D), lambda i, s=s: (i + s * nch, 0))
        for s in range(NSTREAM)
    ] + [pl.BlockSpec((D, E), lambda i: (0, 0))]

    out_specs = []
    out_shape = []
    for _ in range(NSTREAM):
        out_specs += [
            pl.BlockSpec((TILE, E), lambda i: (i, 0)),
            pl.BlockSpec((TILE, 2), lambda i: (i, 0)),
            pl.BlockSpec((TILE, 2), lambda i: (i, 0)),
        ]
        out_shape += [
            jax.ShapeDtypeStruct((M, E), jnp.float32),
            jax.ShapeDtypeStruct((M, 2), jnp.float32),
            jax.ShapeDtypeStruct((M, 2), jnp.int32),
        ]

    outs = pl.pallas_call(
        _router_body,
        grid=(nch,),
        in_specs=in_specs,
        out_specs=out_specs,
        out_shape=out_shape,
    )(*([x2] * NSTREAM), W)

    weights = jnp.concatenate(outs[0::3], axis=0)
    topw = jnp.concatenate(outs[1::3], axis=0)
    tope = jnp.concatenate(outs[2::3], axis=0)
    return (
        weights.reshape(B, S, E),
        topw.reshape(B, S, 2),
        tope.reshape(B, S, 2),
    )
